# Initial kernel scaffold; baseline (speedup 1.0000x reference)
import jax
import jax.numpy as jnp
from jax.experimental import pallas as pl


def kernel(x, edge_index, W_l, b_l, W_r, W_c, b_c):
    raise NotImplementedError("write your pallas kernel here")


# speedup vs baseline: 7.6048x; 7.6048x over previous
import jax
import jax.numpy as jnp
from jax import lax
from jax.experimental import pallas as pl
from jax.experimental.pallas import tpu as pltpu
from jax.experimental.pallas import tpu_sc as plsc

N_NODES = 50000
N_EDGES = 800000
IN_CH = 100
NUM_CLASSES = 16

D_PAD = 128
CNT_COL = 100

NC = 2
NS = 16
RB = 12512
ACC_ROWS = 12544
ZROWS = ACC_ROWS // NS
DUMP = RB
OUT_ROWS = 4 * RB

ES = N_EDGES // NS
E_CH = 2000
INNER = E_CH // 16
NCH_SCAN = ES // E_CH
G = 32
LIST = 2304


def _sc_body(xpad, srch, dsth, zrosh, outh,
             gidx, didx, gst0, dst0, gst1, dst1, gst2, dst2, gst3, dst3,
             rows0, rows1, rows2, rows3, srcv, dstv, acc_sh, semA, semB):
  c = lax.axis_index("c")
  s = lax.axis_index("s")
  eoff = s * ES
  gsts = (gst0, gst1, gst2, gst3)
  dsts = (dst0, dst1, dst2, dst3)
  rows = (rows0, rows1, rows2, rows3)

  def stage(off, gst, dstg):
    for k in range(G // 16):
      gst[pl.ds(k * 16, 16)] = gidx[pl.ds(off + k * 16, 16)]
      dstg[pl.ds(k * 16, 16)] = didx[pl.ds(off + k * 16, 16)]

  def fire(off):
    stage(off, gst0, dst0)
    pltpu.async_copy(xpad.at[gst0], rows0, semA).wait()
    pltpu.sync_copy(rows0, acc_sh.at[dst0], add=True)

  for p in range(2):
    base = (2 * c + p) * RB
    pltpu.sync_copy(zrosh, acc_sh.at[pl.ds(s * ZROWS, ZROWS)])
    plsc.subcore_barrier()

    def chunk_body(j, cnt):
      pltpu.sync_copy(srch.at[pl.ds(eoff + j * E_CH, E_CH)], srcv)
      pltpu.sync_copy(dsth.at[pl.ds(eoff + j * E_CH, E_CH)], dstv)

      def inner(i, cnt):
        dvec = dstv[pl.ds(i * 16, 16)]
        svec = srcv[pl.ds(i * 16, 16)]
        m = (dvec >= base) & (dvec < base + RB)
        mi = m.astype(jnp.int32)
        pos = cnt + plsc.cumsum(mi) - 1
        plsc.store_scatter(gidx, [pos], svec, mask=m)
        plsc.store_scatter(didx, [pos], dvec - base, mask=m)
        return cnt + jnp.sum(mi)

      cnt = lax.fori_loop(0, INNER, inner, cnt)

      nfull = lax.div(cnt, jnp.int32(G))
      nquads = lax.div(nfull, jnp.int32(4))

      def quad_body(q, carry):
        off = q * (4 * G)
        ds = []
        for b in range(4):
          stage(off + b * G, gsts[b], dsts[b])
          ds.append(pltpu.async_copy(xpad.at[gsts[b]], rows[b], semA))
        ss = []
        for b in range(4):
          ds[b].wait()
          ss.append(pltpu.async_copy(rows[b], acc_sh.at[dsts[b]], semB,
                                     add=True))
        for b in range(4):
          ss[b].wait()
        return carry

      lax.fori_loop(0, nquads, quad_body, jnp.int32(0))

      def rem_body(r, carry):
        fire((nquads * 4 + r) * G)
        return carry

      lax.fori_loop(0, nfull - nquads * 4, rem_body, jnp.int32(0))

      rem_off = nfull * G
      for k in range(G // 16):
        gv = gidx[pl.ds(rem_off + k * 16, 16)]
        dv = didx[pl.ds(rem_off + k * 16, 16)]
        gidx[pl.ds(k * 16, 16)] = gv
        didx[pl.ds(k * 16, 16)] = dv
      return cnt - rem_off

    cnt = lax.fori_loop(0, NCH_SCAN, chunk_body, jnp.int32(0))

    zi = jnp.zeros((16,), jnp.int32)
    di = jnp.full((16,), DUMP, jnp.int32)
    for k in range(G // 16):
      gidx[pl.ds(cnt + k * 16, 16)] = zi
      didx[pl.ds(cnt + k * 16, 16)] = di

    @pl.when(cnt > 0)
    def _():
      fire(0)

    plsc.subcore_barrier()

    @pl.when(s < 4)
    def _():
      pltpu.sync_copy(acc_sh.at[pl.ds(s * 3128, 3128)],
                      outh.at[pl.ds(base + s * 3128, 3128)])

    plsc.subcore_barrier()


def _segment_sum_sc(x_pad, src, dst, zeros_tile):
  mesh = plsc.VectorSubcoreMesh(core_axis_name="c", subcore_axis_name="s")
  return pl.kernel(
      _sc_body,
      out_type=jax.ShapeDtypeStruct((OUT_ROWS, D_PAD), jnp.float32),
      mesh=mesh,
      compiler_params=pltpu.CompilerParams(needs_layout_passes=False),
      scratch_types=[
          pltpu.VMEM((LIST,), jnp.int32),
          pltpu.VMEM((LIST,), jnp.int32),
          pltpu.VMEM((G,), jnp.int32),
          pltpu.VMEM((G,), jnp.int32),
          pltpu.VMEM((G,), jnp.int32),
          pltpu.VMEM((G,), jnp.int32),
          pltpu.VMEM((G,), jnp.int32),
          pltpu.VMEM((G,), jnp.int32),
          pltpu.VMEM((G,), jnp.int32),
          pltpu.VMEM((G,), jnp.int32),
          pltpu.VMEM((G, D_PAD), jnp.float32),
          pltpu.VMEM((G, D_PAD), jnp.float32),
          pltpu.VMEM((G, D_PAD), jnp.float32),
          pltpu.VMEM((G, D_PAD), jnp.float32),
          pltpu.VMEM((E_CH,), jnp.int32),
          pltpu.VMEM((E_CH,), jnp.int32),
          pltpu.VMEM_SHARED((ACC_ROWS, D_PAD), jnp.float32),
          pltpu.SemaphoreType.DMA,
          pltpu.SemaphoreType.DMA,
      ],
  )(x_pad, src, dst, zeros_tile)


BM = 2000


def _tc_body(acc_ref, x_ref, wl_ref, bl_ref, wr_ref, wc_ref, bc_ref, out_ref):
  dn = (((1,), (1,)), ((), ()))
  su = acc_ref[:, :IN_CH]
  cnt = acc_ref[:, CNT_COL:CNT_COL + 1]
  mean = su / jnp.maximum(cnt, 1.0)
  h = lax.dot_general(mean, wl_ref[...], dn,
                      preferred_element_type=jnp.float32,
                      precision=lax.Precision.HIGHEST)
  h = h + lax.dot_general(x_ref[...], wr_ref[...], dn,
                          preferred_element_type=jnp.float32,
                          precision=lax.Precision.HIGHEST)
  h = jnp.maximum(h + bl_ref[...], 0.0)
  out_ref[...] = lax.dot_general(h, wc_ref[...], dn,
                                 preferred_element_type=jnp.float32,
                                 precision=lax.Precision.HIGHEST) + bc_ref[...]


def _head_tc(acc, x, W_l, b_l, W_r, W_c, b_c):
  grid = (N_NODES // BM,)
  return pl.pallas_call(
      _tc_body,
      grid=grid,
      in_specs=[
          pl.BlockSpec((BM, D_PAD), lambda i: (i, 0)),
          pl.BlockSpec((BM, IN_CH), lambda i: (i, 0)),
          pl.BlockSpec((IN_CH, IN_CH), lambda i: (0, 0)),
          pl.BlockSpec((1, IN_CH), lambda i: (0, 0)),
          pl.BlockSpec((IN_CH, IN_CH), lambda i: (0, 0)),
          pl.BlockSpec((NUM_CLASSES, IN_CH), lambda i: (0, 0)),
          pl.BlockSpec((1, NUM_CLASSES), lambda i: (0, 0)),
      ],
      out_specs=pl.BlockSpec((BM, NUM_CLASSES), lambda i: (i, 0)),
      out_shape=jax.ShapeDtypeStruct((N_NODES, NUM_CLASSES), jnp.float32),
  )(acc, x, W_l, b_l.reshape(1, IN_CH), W_r, W_c, b_c.reshape(1, NUM_CLASSES))


def kernel(x, edge_index, W_l, b_l, W_r, W_c, b_c):
  src = edge_index[0].astype(jnp.int32)
  dst = edge_index[1].astype(jnp.int32)
  x_pad = jnp.concatenate(
      [x, jnp.ones((N_NODES, 1), jnp.float32),
       jnp.zeros((N_NODES, D_PAD - IN_CH - 1), jnp.float32)], axis=1)
  zeros_tile = jnp.zeros((ZROWS, D_PAD), jnp.float32)
  acc = _segment_sum_sc(x_pad, src, dst, zeros_tile)
  return _head_tc(acc, x, W_l, b_l, W_r, W_c, b_c)


# speedup vs baseline: 9.6001x; 1.2624x over previous
import jax
import jax.numpy as jnp
from jax import lax
from jax.experimental import pallas as pl
from jax.experimental.pallas import tpu as pltpu
from jax.experimental.pallas import tpu_sc as plsc

N_NODES = 50000
N_EDGES = 800000
IN_CH = 100
NUM_CLASSES = 16

D_PAD = 128
CNT_COL = 100

NC = 2
NS = 16
RB = 12512
ACC_ROWS = 12544
ZROWS = ACC_ROWS // NS
DUMP = RB
OUT_ROWS = 4 * RB

ES = N_EDGES // NS
E_CH = 2000
INNER = E_CH // 16
NCH_SCAN = ES // E_CH
G = 32
LIST = 2304


def _sc_body(xpad, srch, dsth, zrosh, outh,
             gidx, didx, gst0, dst0, gst1, dst1, gst2, dst2, gst3, dst3,
             rows0, rows1, rows2, rows3, srcv, dstv, acc_sh, semA, semB):
  c = lax.axis_index("c")
  s = lax.axis_index("s")
  eoff = s * ES
  gsts = (gst0, gst1, gst2, gst3)
  dsts = (dst0, dst1, dst2, dst3)
  rows = (rows0, rows1, rows2, rows3)

  def stage(off, gst, dstg):
    for k in range(G // 16):
      gst[pl.ds(k * 16, 16)] = gidx[pl.ds(off + k * 16, 16)]
      dstg[pl.ds(k * 16, 16)] = didx[pl.ds(off + k * 16, 16)]

  def fire(off):
    stage(off, gst0, dst0)
    pltpu.async_copy(xpad.at[gst0], rows0, semA).wait()
    pltpu.sync_copy(rows0, acc_sh.at[dst0], add=True)

  for p in range(2):
    base = (2 * c + p) * RB
    pltpu.sync_copy(zrosh, acc_sh.at[pl.ds(s * ZROWS, ZROWS)])
    plsc.subcore_barrier()

    def drain_one():
      pltpu.make_async_copy(xpad.at[pl.ds(0, G)], rows0, semB).wait()

    def chunk_body(j, state):
      cnt, nprimed = state
      pltpu.sync_copy(srch.at[pl.ds(eoff + j * E_CH, E_CH)], srcv)
      pltpu.sync_copy(dsth.at[pl.ds(eoff + j * E_CH, E_CH)], dstv)

      def inner(i, cnt):
        dvec = dstv[pl.ds(i * 16, 16)]
        svec = srcv[pl.ds(i * 16, 16)]
        m = (dvec >= base) & (dvec < base + RB)
        mi = m.astype(jnp.int32)
        pos = cnt + plsc.cumsum(mi) - 1
        plsc.store_scatter(gidx, [pos], svec, mask=m)
        plsc.store_scatter(didx, [pos], dvec - base, mask=m)
        return cnt + jnp.sum(mi)

      cnt = lax.fori_loop(0, INNER, inner, cnt)

      nquads = lax.div(cnt, jnp.int32(4 * G))

      def quad_body(q, np_):
        off = q * (4 * G)
        ds = []
        for b in range(4):
          @pl.when(np_ > b)
          def _():
            drain_one()
          stage(off + b * G, gsts[b], dsts[b])
          ds.append(pltpu.async_copy(xpad.at[gsts[b]], rows[b], semA))
        for b in range(4):
          ds[b].wait()
          pltpu.async_copy(rows[b], acc_sh.at[dsts[b]], semB, add=True)
        return jnp.int32(4)

      nprimed = lax.fori_loop(0, nquads, quad_body, nprimed)

      rem_off = nquads * (4 * G)
      for k in range(4 * G // 16):
        gv = gidx[pl.ds(rem_off + k * 16, 16)]
        dv = didx[pl.ds(rem_off + k * 16, 16)]
        gidx[pl.ds(k * 16, 16)] = gv
        didx[pl.ds(k * 16, 16)] = dv
      return (cnt - rem_off, nprimed)

    cnt, nprimed = lax.fori_loop(0, NCH_SCAN, chunk_body,
                                 (jnp.int32(0), jnp.int32(0)))

    def drain_body(r, carry):
      drain_one()
      return carry

    lax.fori_loop(0, nprimed, drain_body, jnp.int32(0))

    zi = jnp.zeros((16,), jnp.int32)
    di = jnp.full((16,), DUMP, jnp.int32)
    for k in range(G // 16):
      gidx[pl.ds(cnt + k * 16, 16)] = zi
      didx[pl.ds(cnt + k * 16, 16)] = di

    ntail = lax.div(cnt + (G - 1), jnp.int32(G))

    def tail_body(r, carry):
      fire(r * G)
      return carry

    lax.fori_loop(0, ntail, tail_body, jnp.int32(0))

    plsc.subcore_barrier()

    @pl.when(s < 4)
    def _():
      pltpu.sync_copy(acc_sh.at[pl.ds(s * 3128, 3128)],
                      outh.at[pl.ds(base + s * 3128, 3128)])

    plsc.subcore_barrier()


def _segment_sum_sc(x_pad, src, dst, zeros_tile):
  mesh = plsc.VectorSubcoreMesh(core_axis_name="c", subcore_axis_name="s")
  return pl.kernel(
      _sc_body,
      out_type=jax.ShapeDtypeStruct((OUT_ROWS, D_PAD), jnp.float32),
      mesh=mesh,
      compiler_params=pltpu.CompilerParams(needs_layout_passes=False),
      scratch_types=[
          pltpu.VMEM((LIST,), jnp.int32),
          pltpu.VMEM((LIST,), jnp.int32),
          pltpu.VMEM((G,), jnp.int32),
          pltpu.VMEM((G,), jnp.int32),
          pltpu.VMEM((G,), jnp.int32),
          pltpu.VMEM((G,), jnp.int32),
          pltpu.VMEM((G,), jnp.int32),
          pltpu.VMEM((G,), jnp.int32),
          pltpu.VMEM((G,), jnp.int32),
          pltpu.VMEM((G,), jnp.int32),
          pltpu.VMEM((G, D_PAD), jnp.float32),
          pltpu.VMEM((G, D_PAD), jnp.float32),
          pltpu.VMEM((G, D_PAD), jnp.float32),
          pltpu.VMEM((G, D_PAD), jnp.float32),
          pltpu.VMEM((E_CH,), jnp.int32),
          pltpu.VMEM((E_CH,), jnp.int32),
          pltpu.VMEM_SHARED((ACC_ROWS, D_PAD), jnp.float32),
          pltpu.SemaphoreType.DMA,
          pltpu.SemaphoreType.DMA,
      ],
  )(x_pad, src, dst, zeros_tile)


BM = 5000


def _tc_body(acc_ref, x_ref, wl_ref, bl_ref, wr_ref, wc_ref, bc_ref, out_ref):
  dn = (((1,), (1,)), ((), ()))
  su = acc_ref[:, :IN_CH]
  cnt = acc_ref[:, CNT_COL:CNT_COL + 1]
  mean = su / jnp.maximum(cnt, 1.0)
  h = lax.dot_general(mean, wl_ref[...], dn,
                      preferred_element_type=jnp.float32,
                      precision=lax.Precision.DEFAULT)
  h = h + lax.dot_general(x_ref[...], wr_ref[...], dn,
                          preferred_element_type=jnp.float32,
                          precision=lax.Precision.DEFAULT)
  h = jnp.maximum(h + bl_ref[...], 0.0)
  out_ref[...] = lax.dot_general(h, wc_ref[...], dn,
                                 preferred_element_type=jnp.float32,
                                 precision=lax.Precision.DEFAULT) + bc_ref[...]


def _head_tc(acc, x, W_l, b_l, W_r, W_c, b_c):
  grid = (N_NODES // BM,)
  return pl.pallas_call(
      _tc_body,
      grid=grid,
      in_specs=[
          pl.BlockSpec((BM, D_PAD), lambda i: (i, 0)),
          pl.BlockSpec((BM, IN_CH), lambda i: (i, 0)),
          pl.BlockSpec((IN_CH, IN_CH), lambda i: (0, 0)),
          pl.BlockSpec((1, IN_CH), lambda i: (0, 0)),
          pl.BlockSpec((IN_CH, IN_CH), lambda i: (0, 0)),
          pl.BlockSpec((NUM_CLASSES, IN_CH), lambda i: (0, 0)),
          pl.BlockSpec((1, NUM_CLASSES), lambda i: (0, 0)),
      ],
      out_specs=pl.BlockSpec((BM, NUM_CLASSES), lambda i: (i, 0)),
      out_shape=jax.ShapeDtypeStruct((N_NODES, NUM_CLASSES), jnp.float32),
  )(acc, x, W_l, b_l.reshape(1, IN_CH), W_r, W_c, b_c.reshape(1, NUM_CLASSES))


def kernel(x, edge_index, W_l, b_l, W_r, W_c, b_c):
  src = edge_index[0].astype(jnp.int32)
  dst = edge_index[1].astype(jnp.int32)
  x_pad = jnp.concatenate(
      [x, jnp.ones((N_NODES, 1), jnp.float32),
       jnp.zeros((N_NODES, D_PAD - IN_CH - 1), jnp.float32)], axis=1)
  zeros_tile = jnp.zeros((ZROWS, D_PAD), jnp.float32)
  acc = _segment_sum_sc(x_pad, src, dst, zeros_tile)
  return _head_tc(acc, x, W_l, b_l, W_r, W_c, b_c)


# speedup vs baseline: 10.5785x; 1.1019x over previous
import jax
import jax.numpy as jnp
from jax import lax
from jax.experimental import pallas as pl
from jax.experimental.pallas import tpu as pltpu
from jax.experimental.pallas import tpu_sc as plsc

N_NODES = 50000
N_EDGES = 800000
IN_CH = 100
NUM_CLASSES = 16

D_PAD = 128
CNT_COL = 100

NC = 2
NS = 16
RB = 12512
ACC_ROWS = 12544
ZROWS = ACC_ROWS // NS
DUMP = RB
OUT_ROWS = 4 * RB

ES = N_EDGES // NS
E_CH = 2000
INNER = E_CH // 16
NCH_SCAN = ES // E_CH
G = 32
LIST = 2304


def _sc_body(xpad, srch, dsth, zrosh, outh,
             gidx, didx, gst0, dst0, gst1, dst1, gst2, dst2, gst3, dst3,
             rows0, rows1, rows2, rows3, srcv, dstv, srcv2, dstv2, acc_sh,
             semA, semB, semC):
  c = lax.axis_index("c")
  s = lax.axis_index("s")
  eoff = s * ES
  gsts = (gst0, gst1, gst2, gst3)
  dsts = (dst0, dst1, dst2, dst3)
  rows = (rows0, rows1, rows2, rows3)

  def stage(off, gst, dstg):
    for k in range(G // 16):
      gst[pl.ds(k * 16, 16)] = gidx[pl.ds(off + k * 16, 16)]
      dstg[pl.ds(k * 16, 16)] = didx[pl.ds(off + k * 16, 16)]

  def fire(off):
    stage(off, gst0, dst0)
    pltpu.async_copy(xpad.at[gst0], rows0, semA).wait()
    pltpu.sync_copy(rows0, acc_sh.at[dst0], add=True)

  for p in range(2):
    base = (2 * c + p) * RB
    pltpu.sync_copy(zrosh, acc_sh.at[pl.ds(s * ZROWS, ZROWS)])
    plsc.subcore_barrier()

    def drain_one():
      pltpu.make_async_copy(xpad.at[pl.ds(0, G)], rows0, semB).wait()

    def chunk_body(j, state, sv, dv, svn, dvn, last):
      cnt, nprimed = state
      pltpu.make_async_copy(srch.at[pl.ds(0, E_CH)], sv, semC).wait()
      pltpu.make_async_copy(dsth.at[pl.ds(0, E_CH)], dv, semC).wait()
      if not last:
        pltpu.async_copy(srch.at[pl.ds(eoff + (j + 1) * E_CH, E_CH)], svn,
                         semC)
        pltpu.async_copy(dsth.at[pl.ds(eoff + (j + 1) * E_CH, E_CH)], dvn,
                         semC)

      def inner(i, cnt):
        dvec = dv[pl.ds(i * 16, 16)]
        svec = sv[pl.ds(i * 16, 16)]
        m = (dvec >= base) & (dvec < base + RB)
        mi = m.astype(jnp.int32)
        pos = cnt + plsc.cumsum(mi) - 1
        plsc.store_scatter(gidx, [pos], svec, mask=m)
        plsc.store_scatter(didx, [pos], dvec - base, mask=m)
        return cnt + jnp.sum(mi)

      cnt = lax.fori_loop(0, INNER, inner, cnt)

      nquads = lax.div(cnt, jnp.int32(4 * G))

      def quad_body(q, np_):
        off = q * (4 * G)
        ds = []
        for b in range(4):
          @pl.when(np_ > b)
          def _():
            drain_one()
          stage(off + b * G, gsts[b], dsts[b])
          ds.append(pltpu.async_copy(xpad.at[gsts[b]], rows[b], semA))
        for b in range(4):
          ds[b].wait()
          pltpu.async_copy(rows[b], acc_sh.at[dsts[b]], semB, add=True)
        return jnp.int32(4)

      nprimed = lax.fori_loop(0, nquads, quad_body, nprimed)

      rem_off = nquads * (4 * G)
      for k in range(4 * G // 16):
        gv = gidx[pl.ds(rem_off + k * 16, 16)]
        dv = didx[pl.ds(rem_off + k * 16, 16)]
        gidx[pl.ds(k * 16, 16)] = gv
        didx[pl.ds(k * 16, 16)] = dv
      return (cnt - rem_off, nprimed)

    pltpu.async_copy(srch.at[pl.ds(eoff, E_CH)], srcv, semC)
    pltpu.async_copy(dsth.at[pl.ds(eoff, E_CH)], dstv, semC)

    def pair_chunks(jj, state):
      state = chunk_body(2 * jj, state, srcv, dstv, srcv2, dstv2, False)
      return chunk_body(2 * jj + 1, state, srcv2, dstv2, srcv, dstv, False)

    state = lax.fori_loop(0, (NCH_SCAN - 1) // 2, pair_chunks,
                          (jnp.int32(0), jnp.int32(0)))
    cnt, nprimed = chunk_body(jnp.int32(NCH_SCAN - 1), state,
                              srcv, dstv, srcv2, dstv2, True)

    def drain_body(r, carry):
      drain_one()
      return carry

    lax.fori_loop(0, nprimed, drain_body, jnp.int32(0))

    zi = jnp.zeros((16,), jnp.int32)
    di = jnp.full((16,), DUMP, jnp.int32)
    for k in range(G // 16):
      gidx[pl.ds(cnt + k * 16, 16)] = zi
      didx[pl.ds(cnt + k * 16, 16)] = di

    ntail = lax.div(cnt + (G - 1), jnp.int32(G))

    def tail_body(r, carry):
      fire(r * G)
      return carry

    lax.fori_loop(0, ntail, tail_body, jnp.int32(0))

    plsc.subcore_barrier()

    @pl.when(s < 4)
    def _():
      pltpu.sync_copy(acc_sh.at[pl.ds(s * 3128, 3128)],
                      outh.at[pl.ds(base + s * 3128, 3128)])

    plsc.subcore_barrier()


def _segment_sum_sc(x_pad, src, dst, zeros_tile):
  mesh = plsc.VectorSubcoreMesh(core_axis_name="c", subcore_axis_name="s")
  return pl.kernel(
      _sc_body,
      out_type=jax.ShapeDtypeStruct((OUT_ROWS, D_PAD), jnp.float32),
      mesh=mesh,
      compiler_params=pltpu.CompilerParams(needs_layout_passes=False),
      scratch_types=[
          pltpu.VMEM((LIST,), jnp.int32),
          pltpu.VMEM((LIST,), jnp.int32),
          pltpu.VMEM((G,), jnp.int32),
          pltpu.VMEM((G,), jnp.int32),
          pltpu.VMEM((G,), jnp.int32),
          pltpu.VMEM((G,), jnp.int32),
          pltpu.VMEM((G,), jnp.int32),
          pltpu.VMEM((G,), jnp.int32),
          pltpu.VMEM((G,), jnp.int32),
          pltpu.VMEM((G,), jnp.int32),
          pltpu.VMEM((G, D_PAD), jnp.float32),
          pltpu.VMEM((G, D_PAD), jnp.float32),
          pltpu.VMEM((G, D_PAD), jnp.float32),
          pltpu.VMEM((G, D_PAD), jnp.float32),
          pltpu.VMEM((E_CH,), jnp.int32),
          pltpu.VMEM((E_CH,), jnp.int32),
          pltpu.VMEM((E_CH,), jnp.int32),
          pltpu.VMEM((E_CH,), jnp.int32),
          pltpu.VMEM_SHARED((ACC_ROWS, D_PAD), jnp.float32),
          pltpu.SemaphoreType.DMA,
          pltpu.SemaphoreType.DMA,
          pltpu.SemaphoreType.DMA,
      ],
  )(x_pad, src, dst, zeros_tile)


BM = 5000


def _tc_body(acc_ref, x_ref, wl_ref, bl_ref, wr_ref, wc_ref, bc_ref, out_ref):
  dn = (((1,), (1,)), ((), ()))
  su = acc_ref[:, :IN_CH]
  cnt = acc_ref[:, CNT_COL:CNT_COL + 1]
  mean = su / jnp.maximum(cnt, 1.0)
  h = lax.dot_general(mean, wl_ref[...], dn,
                      preferred_element_type=jnp.float32,
                      precision=lax.Precision.DEFAULT)
  h = h + lax.dot_general(x_ref[...], wr_ref[...], dn,
                          preferred_element_type=jnp.float32,
                          precision=lax.Precision.DEFAULT)
  h = jnp.maximum(h + bl_ref[...], 0.0)
  out_ref[...] = lax.dot_general(h, wc_ref[...], dn,
                                 preferred_element_type=jnp.float32,
                                 precision=lax.Precision.DEFAULT) + bc_ref[...]


def _head_tc(acc, x, W_l, b_l, W_r, W_c, b_c):
  grid = (N_NODES // BM,)
  return pl.pallas_call(
      _tc_body,
      grid=grid,
      in_specs=[
          pl.BlockSpec((BM, D_PAD), lambda i: (i, 0)),
          pl.BlockSpec((BM, IN_CH), lambda i: (i, 0)),
          pl.BlockSpec((IN_CH, IN_CH), lambda i: (0, 0)),
          pl.BlockSpec((1, IN_CH), lambda i: (0, 0)),
          pl.BlockSpec((IN_CH, IN_CH), lambda i: (0, 0)),
          pl.BlockSpec((NUM_CLASSES, IN_CH), lambda i: (0, 0)),
          pl.BlockSpec((1, NUM_CLASSES), lambda i: (0, 0)),
      ],
      out_specs=pl.BlockSpec((BM, NUM_CLASSES), lambda i: (i, 0)),
      out_shape=jax.ShapeDtypeStruct((N_NODES, NUM_CLASSES), jnp.float32),
  )(acc, x, W_l, b_l.reshape(1, IN_CH), W_r, W_c, b_c.reshape(1, NUM_CLASSES))


def kernel(x, edge_index, W_l, b_l, W_r, W_c, b_c):
  src = edge_index[0].astype(jnp.int32)
  dst = edge_index[1].astype(jnp.int32)
  x_pad = jnp.concatenate(
      [x, jnp.ones((N_NODES, 1), jnp.float32),
       jnp.zeros((N_NODES, D_PAD - IN_CH - 1), jnp.float32)], axis=1)
  zeros_tile = jnp.zeros((ZROWS, D_PAD), jnp.float32)
  acc = _segment_sum_sc(x_pad, src, dst, zeros_tile)
  return _head_tc(acc, x, W_l, b_l, W_r, W_c, b_c)
